# baseline (speedup 1.0000x reference)
import jax
import jax.numpy as jnp
from jax.experimental import pallas as pl
from jax.experimental.pallas import tpu as pltpu

N = 10000
D = 128
BM = 200
NB = N // BM
KEEP = 10
NK = NB - KEEP

F8 = jnp.float8_e4m3fn


def _fused_kernel(
    a_ref, ht_ref, w0_ref, b0_ref, w1_ref, b1_ref,
    o_ref, q_ref,
    h1_ref, xw0_ref, xw1_ref, stage_ref, keep_ref, sem,
):
    p = pl.program_id(0)
    i = pl.program_id(1)

    @pl.when(p == 0)
    def _phase0():
        @pl.when(i == 0)
        def _pre():
            xw0_ref[...] = jnp.dot(
                ht_ref[...].astype(jnp.bfloat16),
                w0_ref[...].astype(jnp.bfloat16),
                preferred_element_type=jnp.float32,
            ).astype(jnp.bfloat16)

        a = a_ref[...]
        part = jnp.dot(
            a.astype(jnp.bfloat16),
            xw0_ref[...],
            preferred_element_type=jnp.float32,
        )
        h1_ref[pl.ds(i * BM, BM), :] = jnp.maximum(part + b0_ref[...], 0.0)

        qa = a.astype(F8)
        slot = jax.lax.rem(i, 2)

        @pl.when(i < NK)
        def _spill():
            @pl.when(i >= 2)
            def _wait_prev():
                pltpu.make_async_copy(
                    stage_ref.at[slot],
                    q_ref.at[pl.ds((i - 2) * BM, BM), :],
                    sem.at[slot],
                ).wait()

            stage_ref[slot] = qa
            pltpu.make_async_copy(
                stage_ref.at[slot],
                q_ref.at[pl.ds(i * BM, BM), :],
                sem.at[slot],
            ).start()

        @pl.when(i >= NK)
        def _retain():
            keep_ref[pl.ds((i - NK) * BM, BM), :] = qa

    @pl.when(p == 1)
    def _phase1():
        @pl.when(i == 0)
        def _pre1():
            pltpu.make_async_copy(
                stage_ref.at[(NK - 2) % 2],
                q_ref.at[pl.ds((NK - 2) * BM, BM), :],
                sem.at[(NK - 2) % 2],
            ).wait()
            pltpu.make_async_copy(
                stage_ref.at[(NK - 1) % 2],
                q_ref.at[pl.ds((NK - 1) * BM, BM), :],
                sem.at[(NK - 1) % 2],
            ).wait()
            xw1_ref[...] = jnp.dot(
                h1_ref[...].astype(jnp.bfloat16),
                w1_ref[...].astype(jnp.bfloat16),
                preferred_element_type=jnp.float32,
            ).astype(F8)
            pltpu.make_async_copy(
                q_ref.at[pl.ds(0, BM), :], stage_ref.at[0], sem.at[0]
            ).start()
            pltpu.make_async_copy(
                q_ref.at[pl.ds(BM, BM), :], stage_ref.at[1], sem.at[1]
            ).start()

        slot = jax.lax.rem(i, 2)

        @pl.when(i < NK)
        def _from_hbm():
            pltpu.make_async_copy(
                q_ref.at[pl.ds(i * BM, BM), :],
                stage_ref.at[slot],
                sem.at[slot],
            ).wait()
            qa = stage_ref[slot]
            part = jax.lax.dot_general(
                qa, xw1_ref[...],
                (((1,), (0,)), ((), ())),
                preferred_element_type=jnp.float32,
            )
            o_ref[...] = jnp.maximum(part + b1_ref[...], 0.0)

            @pl.when(i + 2 < NK)
            def _prefetch():
                pltpu.make_async_copy(
                    q_ref.at[pl.ds((i + 2) * BM, BM), :],
                    stage_ref.at[slot],
                    sem.at[slot],
                ).start()

        @pl.when(i >= NK)
        def _from_vmem():
            qa = keep_ref[pl.ds((i - NK) * BM, BM), :]
            part = jax.lax.dot_general(
                qa, xw1_ref[...],
                (((1,), (0,)), ((), ())),
                preferred_element_type=jnp.float32,
            )
            o_ref[...] = jnp.maximum(part + b1_ref[...], 0.0)


def kernel(ht, adjs, W0, b0, W1, b1):
    out, _ = pl.pallas_call(
        _fused_kernel,
        grid=(2, NB),
        in_specs=[
            pl.BlockSpec((BM, N), lambda p, i: (jnp.where(p == 0, i, NB - 1), 0)),
            pl.BlockSpec((N, D), lambda p, i: (0, 0)),
            pl.BlockSpec((D, D), lambda p, i: (0, 0)),
            pl.BlockSpec((1, D), lambda p, i: (0, 0)),
            pl.BlockSpec((D, D), lambda p, i: (0, 0)),
            pl.BlockSpec((1, D), lambda p, i: (0, 0)),
        ],
        out_specs=[
            pl.BlockSpec((BM, D), lambda p, i: (jnp.where(p == 0, 0, i), 0)),
            pl.BlockSpec(memory_space=pl.ANY),
        ],
        out_shape=[
            jax.ShapeDtypeStruct((N, D), jnp.float32),
            jax.ShapeDtypeStruct((NK * BM, N), F8),
        ],
        scratch_shapes=[
            pltpu.VMEM((N, D), jnp.float32),
            pltpu.VMEM((N, D), jnp.bfloat16),
            pltpu.VMEM((N, D), F8),
            pltpu.VMEM((2, BM, N), F8),
            pltpu.VMEM((KEEP * BM, N), F8),
            pltpu.SemaphoreType.DMA((2,)),
        ],
        compiler_params=pltpu.CompilerParams(
            dimension_semantics=("arbitrary", "arbitrary"),
        ),
    )(adjs, ht, W0, b0.reshape(1, D), W1, b1.reshape(1, D))
    return out


# speedup vs baseline: 1.0804x; 1.0804x over previous
import jax
import jax.numpy as jnp
from jax.experimental import pallas as pl
from jax.experimental.pallas import tpu as pltpu

N = 10000
D = 128
BM = 400
BM2 = 1600

F8 = jnp.float8_e4m3fn


def _layer1_kernel(a_ref, x_ref, w_ref, b_ref, o_ref, q_ref, xw_ref):
    i = pl.program_id(0)

    @pl.when(i == 0)
    def _pre():
        xw_ref[...] = jnp.dot(
            x_ref[...].astype(jnp.bfloat16),
            w_ref[...].astype(jnp.bfloat16),
            preferred_element_type=jnp.float32,
        ).astype(jnp.bfloat16)

    a = a_ref[...]
    q_ref[...] = a.astype(F8)
    part = jnp.dot(
        a.astype(jnp.bfloat16),
        xw_ref[...],
        preferred_element_type=jnp.float32,
    )
    o_ref[...] = jnp.maximum(part + b_ref[...], 0.0)


def _layer2_kernel(q_ref, x_ref, w_ref, b_ref, o_ref, xw_ref):
    i = pl.program_id(0)

    @pl.when(i == 0)
    def _pre():
        xw_ref[...] = jnp.dot(
            x_ref[...].astype(jnp.bfloat16),
            w_ref[...].astype(jnp.bfloat16),
            preferred_element_type=jnp.float32,
        ).astype(F8)

    part = jax.lax.dot_general(
        q_ref[...],
        xw_ref[...],
        (((1,), (0,)), ((), ())),
        preferred_element_type=jnp.float32,
    )
    o_ref[...] = jnp.maximum(part + b_ref[...], 0.0)


def _layer1(adjs, x, w, b):
    return pl.pallas_call(
        _layer1_kernel,
        grid=(N // BM,),
        in_specs=[
            pl.BlockSpec((BM, N), lambda i: (i, 0)),
            pl.BlockSpec((N, D), lambda i: (0, 0)),
            pl.BlockSpec((D, D), lambda i: (0, 0)),
            pl.BlockSpec((1, D), lambda i: (0, 0)),
        ],
        out_specs=[
            pl.BlockSpec((BM, D), lambda i: (i, 0)),
            pl.BlockSpec((BM, N), lambda i: (i, 0)),
        ],
        out_shape=[
            jax.ShapeDtypeStruct((N, D), jnp.float32),
            jax.ShapeDtypeStruct((N, N), F8),
        ],
        scratch_shapes=[pltpu.VMEM((N, D), jnp.bfloat16)],
        compiler_params=pltpu.CompilerParams(
            dimension_semantics=("arbitrary",),
        ),
    )(adjs, x, w, b)


def _layer2(q, x, w, b):
    return pl.pallas_call(
        _layer2_kernel,
        grid=(pl.cdiv(N, BM2),),
        in_specs=[
            pl.BlockSpec((BM2, N), lambda i: (i, 0)),
            pl.BlockSpec((N, D), lambda i: (0, 0)),
            pl.BlockSpec((D, D), lambda i: (0, 0)),
            pl.BlockSpec((1, D), lambda i: (0, 0)),
        ],
        out_specs=pl.BlockSpec((BM2, D), lambda i: (i, 0)),
        out_shape=jax.ShapeDtypeStruct((N, D), jnp.float32),
        scratch_shapes=[pltpu.VMEM((N, D), F8)],
        compiler_params=pltpu.CompilerParams(
            dimension_semantics=("arbitrary",),
        ),
    )(q, x, w, b)


def kernel(ht, adjs, W0, b0, W1, b1):
    h1, q = _layer1(adjs, ht, W0, b0.reshape(1, D))
    h2 = _layer2(q, h1, W1, b1.reshape(1, D))
    return h2
